# Initial kernel scaffold; baseline (speedup 1.0000x reference)
import jax
import jax.numpy as jnp
from jax.experimental import pallas as pl


def kernel(src, dest, edge_attr, u, batch, W1, b1, W2, b2):
    raise NotImplementedError("write your pallas kernel here")


# speedup vs baseline: 7.6522x; 7.6522x over previous
import functools

import jax
import jax.numpy as jnp
from jax import lax
from jax.experimental import pallas as pl
from jax.experimental.pallas import tpu as pltpu
from jax.experimental.pallas import tpu_sc as plsc

_LANES = 128
_SC_WORKERS = 32
_SC_CHUNK = 10000
_SC_VECLEN = 16


def _sc_gather(u_flat, batch):
    E = batch.shape[0]
    V = u_flat.shape[0]
    per_w = E // _SC_WORKERS
    n_chunks = per_w // _SC_CHUNK
    mesh = plsc.VectorSubcoreMesh(core_axis_name="c", subcore_axis_name="s")

    @functools.partial(
        pl.kernel,
        mesh=mesh,
        out_type=jax.ShapeDtypeStruct((E,), jnp.float32),
        scratch_types=[
            pltpu.VMEM((V,), jnp.float32),
            pltpu.VMEM((_SC_CHUNK,), jnp.int32),
            pltpu.VMEM((_SC_CHUNK,), jnp.float32),
        ],
        compiler_params=pltpu.CompilerParams(needs_layout_passes=False),
    )
    def gather_kernel(u_hbm, idx_hbm, out_hbm, u_v, idx_v, out_v):
        wid = lax.axis_index("s") * 2 + lax.axis_index("c")
        base = wid * per_w
        pltpu.sync_copy(u_hbm, u_v)

        def chunk_body(c, carry):
            off = base + c * _SC_CHUNK
            pltpu.sync_copy(idx_hbm.at[pl.ds(off, _SC_CHUNK)], idx_v)

            def vec_body(n, carry2):
                iv = idx_v[pl.ds(n * _SC_VECLEN, _SC_VECLEN)]
                out_v[pl.ds(n * _SC_VECLEN, _SC_VECLEN)] = plsc.load_gather(
                    u_v, [iv])
                return carry2

            lax.fori_loop(0, _SC_CHUNK // _SC_VECLEN, vec_body, 0, unroll=8)
            pltpu.sync_copy(out_v, out_hbm.at[pl.ds(off, _SC_CHUNK)])
            return carry

        lax.fori_loop(0, n_chunks, chunk_body, 0)

    return gather_kernel(u_flat, batch)


def _mlp_body(s_ref, d_ref, e_ref, g_ref, w1_ref, b1_ref, w2_ref, b2_ref,
              o_ref):
    x = jnp.concatenate(
        [s_ref[...], d_ref[...], e_ref[...], g_ref[...]], axis=1)
    xb = x.astype(jnp.bfloat16)
    h = lax.dot_general(xb, w1_ref[...], (((1,), (0,)), ((), ())),
                        preferred_element_type=jnp.float32)
    h = jnp.maximum(h + b1_ref[...], 0.0).astype(jnp.bfloat16)
    o = lax.dot_general(h, w2_ref[...], (((1,), (0,)), ((), ())),
                        preferred_element_type=jnp.float32)
    o_ref[...] = o + b2_ref[...]


def _tc_mlp(s2, d2, e2, g2, W1, b1, W2, b2):
    M = s2.shape[0]
    F, H = W1.shape
    K = W2.shape[1]
    R = 1000

    eye = jnp.eye(_LANES, dtype=jnp.float32)
    w1p = (W1[:, None, :, None] * eye[None, :, None, :]).reshape(
        F * _LANES, H * _LANES).astype(jnp.bfloat16)
    w2p = (W2[:, None, None, :] * eye[None, :, :, None]).reshape(
        H * _LANES, K * _LANES).astype(jnp.bfloat16)
    b1eff = jnp.repeat(b1, _LANES)[None, :]
    b2eff = jnp.tile(b2, _LANES)[None, :]

    block = lambda r, c: pl.BlockSpec((r, c), lambda i: (i, 0))
    fixed = lambda r, c: pl.BlockSpec((r, c), lambda i: (0, 0))
    return pl.pallas_call(
        _mlp_body,
        grid=(M // R,),
        in_specs=[
            block(R, _LANES), block(R, _LANES),
            block(R, _LANES), block(R, _LANES),
            fixed(F * _LANES, H * _LANES), fixed(1, H * _LANES),
            fixed(H * _LANES, K * _LANES), fixed(1, K * _LANES),
        ],
        out_specs=block(R, K * _LANES),
        out_shape=jax.ShapeDtypeStruct((M, K * _LANES), jnp.float32),
        compiler_params=pltpu.CompilerParams(
            dimension_semantics=("parallel",)),
    )(s2, d2, e2, g2, w1p, b1eff, w2p, b2eff)


def kernel(src, dest, edge_attr, u, batch, W1, b1, W2, b2):
    E = src.shape[0]
    K = W2.shape[1]
    M = E // _LANES
    ug = _sc_gather(u.reshape(-1), batch)
    out = _tc_mlp(
        src.reshape(M, _LANES), dest.reshape(M, _LANES),
        edge_attr.reshape(M, _LANES), ug.reshape(M, _LANES),
        W1, b1, W2, b2)
    return out.reshape(E, K)
